# baseline (speedup 1.0000x reference)
import jax
import jax.numpy as jnp
from jax import lax
from jax.experimental import pallas as pl
from jax.experimental.pallas import tpu as pltpu
from jax.experimental.pallas import tpu_sc as plsc

N_USERS = 25000
N_ITEMS = 25000
N_NODES = N_USERS + N_ITEMS
D = 64
E = 800000
HALF = N_NODES // 2
N_SUB = 16
N_W = 32
ROWS_PER_SUB = 1563
ACC_ROWS = N_SUB * ROWS_PER_SUB
DUMMY_ROW = 25004
K = 96
W_CHUNKS = 264
W_EDGES = W_CHUNKS * K
E_PAD = N_W * W_EDGES
CAP_BLOCKS = W_CHUNKS + 2
SUB_CAP = CAP_BLOCKS * K

RESIDUAL_COFF = 0.1

P0 = 7.8579951959925864
P1 = -22.224249412937251
P2 = 29.508311855171733
P3 = -24.651553647411124
P4 = 14.776720780182517
P5 = -6.8931074954875129


def _prefix_incl(mask):
    lanes = lax.iota(jnp.int32, 16)
    p = jnp.where(mask, 1, 0)
    for sh in (1, 2, 4, 8):
        shifted = p.at[(lanes - sh) & 15].get(mode="promise_in_bounds")
        p = p + jnp.where(lanes >= sh, shifted, 0)
    return p


def _hsum_bcast(p):
    lanes = lax.iota(jnp.int32, 16)
    for sh in (8, 4, 2, 1):
        idx = (lanes + sh) & 15
        p = p + p.at[idx].get(mode="promise_in_bounds")
    return p


def _l2norm(x, eps=1e-12):
    n = jnp.linalg.norm(x, axis=-1, keepdims=True)
    return x / jnp.maximum(n, eps)


RS = 8


def _route_body(srci, dsti, ev, rsrc, rdst, rev, counts, *scr):
    slots = [tuple(scr[i * 5:(i + 1) * 5]) for i in range(RS)]
    cnt_v = scr[5 * RS]
    sem_in = scr[5 * RS + 1:5 * RS + 1 + RS]
    sem_sc = scr[5 * RS + 1 + RS:5 * RS + 1 + 2 * RS]
    c = lax.axis_index("c")
    s = lax.axis_index("s")
    w = s * 2 + c
    lanes = lax.iota(jnp.int32, 16)
    base0 = w * SUB_CAP
    base1 = (N_W + w) * SUB_CAP
    trash0 = base0 + SUB_CAP - 1
    trash1 = base1 + SUB_CAP - 1

    def issue_loads(j, b):
        in_s, in_d, in_e, _, _ = slots[b]
        base = w * W_EDGES + j * K
        pltpu.async_copy(srci.at[pl.ds(base, K)], in_s, sem_in[b])
        pltpu.async_copy(dsti.at[pl.ds(base, K)], in_d, sem_in[b])
        pltpu.async_copy(ev.at[pl.ds(base, K)], in_e, sem_in[b])

    def wait_loads(b):
        in_s, in_d, in_e, _, _ = slots[b]
        pltpu.make_async_copy(srci.at[pl.ds(0, K)], in_s, sem_in[b]).wait()
        pltpu.make_async_copy(dsti.at[pl.ds(0, K)], in_d, sem_in[b]).wait()
        pltpu.make_async_copy(ev.at[pl.ds(0, K)], in_e, sem_in[b]).wait()

    def issue_scatters(b):
        in_s, in_d, in_e, idx0, idx1 = slots[b]
        pltpu.async_copy(in_s, rsrc.at[idx0], sem_sc[b])
        pltpu.async_copy(in_d, rdst.at[idx0], sem_sc[b])
        pltpu.async_copy(in_e, rev.at[idx0], sem_sc[b])
        pltpu.async_copy(in_s, rsrc.at[idx1], sem_sc[b])
        pltpu.async_copy(in_d, rdst.at[idx1], sem_sc[b])
        pltpu.async_copy(in_e, rev.at[idx1], sem_sc[b])

    def drain_scatters(b):
        in_s, in_d, in_e, idx0, idx1 = slots[b]
        pltpu.make_async_copy(in_s, rsrc.at[idx0], sem_sc[b]).wait()
        pltpu.make_async_copy(in_d, rdst.at[idx0], sem_sc[b]).wait()
        pltpu.make_async_copy(in_e, rev.at[idx0], sem_sc[b]).wait()
        pltpu.make_async_copy(in_s, rsrc.at[idx1], sem_sc[b]).wait()
        pltpu.make_async_copy(in_d, rdst.at[idx1], sem_sc[b]).wait()
        pltpu.make_async_copy(in_e, rev.at[idx1], sem_sc[b]).wait()

    def compute_idx(b, cur0, cur1):
        in_s, in_d, in_e, idx0, idx1 = slots[b]
        for g in range(K // 16):
            sv = in_s[pl.ds(g * 16, 16)]
            m0 = sv < HALF
            pi0 = _prefix_incl(m0)
            n0 = pi0[15]
            pos0 = jnp.where(m0, base0 + cur0 + pi0 - 1, trash0)
            pos1 = jnp.where(m0, trash1, base1 + cur1 + lanes - pi0)
            idx0[pl.ds(g * 16, 16)] = pos0
            idx1[pl.ds(g * 16, 16)] = pos1
            cur0 = cur0 + n0
            cur1 = cur1 + (16 - n0)
        return cur0, cur1

    def step(j, b, cur0, cur1, b_free, do_drain, do_loads):
        wait_loads(b)
        cur0, cur1 = compute_idx(b, cur0, cur1)
        issue_scatters(b)
        if do_drain:
            drain_scatters(b_free)
        if do_loads:
            issue_loads(j + 2, b_free)
        return cur0, cur1

    cur0 = jnp.int32(0)
    cur1 = jnp.int32(0)
    issue_loads(0, 0)
    issue_loads(1, 1)
    for j in range(RS - 2):
        cur0, cur1 = step(j, j % RS, cur0, cur1, (j + 2) % RS, False, True)

    def main_body(i, carry):
        cur0, cur1 = carry
        for b8 in range(RS):
            j = (RS - 2) + i * RS + b8
            b = (RS - 2 + b8) % RS
            cur0, cur1 = step(j, b, cur0, cur1, b8 % RS, True, True)
        return (cur0, cur1)

    n_main = (W_CHUNKS - (RS - 2) - 2) // RS
    cur0, cur1 = lax.fori_loop(0, n_main, main_body, (cur0, cur1))

    for k in range(2):
        j = (RS - 2) + n_main * RS + k
        b = j % RS
        cur0, cur1 = step(j, b, cur0, cur1, (j + 2) % RS, True, False)
    for k in range(RS - 2):
        drain_scatters((W_CHUNKS - (RS - 2) + k) % RS)

    ones = jnp.zeros((16,), jnp.int32)
    cnt_v[pl.ds(0, 16)] = ones + cur0
    pltpu.sync_copy(cnt_v, counts.at[0, w])
    cnt_v[pl.ds(0, 16)] = ones + cur1
    pltpu.sync_copy(cnt_v, counts.at[1, w])


def _edge_body(emb, rsrc, rdst, rev, counts, zrows, out,
               sidx0, didx0, ev0, srows0, drows0,
               sidx1, didx1, ev1, srows1, drows1,
               scat_v, cnt_v, acc, sa0, sb0, sa1, sb1):
    c = lax.axis_index("c")
    s = lax.axis_index("s")
    bufs = ((sidx0, didx0, ev0, srows0, drows0, sa0, sb0),
            (sidx1, didx1, ev1, srows1, drows1, sa1, sb1))

    pltpu.sync_copy(zrows, acc.at[pl.ds(s * ROWS_PER_SUB, ROWS_PER_SUB)])
    plsc.subcore_barrier()

    def issue(w, j, buf):
        sidx_v, didx_v, ev_v, srows, drows, sem_a, sem_b = buf
        base = (c * N_W + w) * SUB_CAP + j * K
        pltpu.sync_copy(rsrc.at[pl.ds(base, K)], sidx_v)
        pltpu.sync_copy(rdst.at[pl.ds(base, K)], didx_v)
        pltpu.sync_copy(rev.at[pl.ds(base, K)], ev_v)
        for g in range(K // 16):
            v = sidx_v[pl.ds(g * 16, 16)]
            sidx_v[pl.ds(g * 16, 16)] = jnp.minimum(
                jnp.maximum(v, 0), N_NODES - 1)
            u = didx_v[pl.ds(g * 16, 16)]
            didx_v[pl.ds(g * 16, 16)] = jnp.minimum(
                jnp.maximum(u, 0), N_NODES - 1)
        pltpu.async_copy(emb.at[sidx_v], srows, sem_a)
        pltpu.async_copy(emb.at[didx_v], drows, sem_b)

    def drain(buf):
        sidx_v, didx_v, ev_v, srows, drows, sem_a, sem_b = buf
        pltpu.make_async_copy(emb.at[sidx_v], srows, sem_a).wait()
        pltpu.make_async_copy(emb.at[didx_v], drows, sem_b).wait()

    def compute_scatter(buf, j, cnt):
        sidx_v, didx_v, ev_v, srows, drows, sem_a, sem_b = buf

        def group_body(g, _):
            evg = ev_v[pl.ds(g * 16, 16)]
            for lane in range(16):
                e = g * 16 + lane
                a0 = srows[e, pl.ds(0, 16)]
                a1 = srows[e, pl.ds(16, 16)]
                a2 = srows[e, pl.ds(32, 16)]
                a3 = srows[e, pl.ds(48, 16)]
                b0 = drows[e, pl.ds(0, 16)]
                b1 = drows[e, pl.ds(16, 16)]
                b2 = drows[e, pl.ds(32, 16)]
                b3 = drows[e, pl.ds(48, 16)]
                p = a0 * b0 + a1 * b1 + a2 * b2 + a3 * b3
                dot = _hsum_bcast(p)
                cm = dot * (1.0 / 64.0)
                wgt = ((((P5 * cm + P4) * cm + P3) * cm + P2) * cm + P1) * cm + P0
                wgt = wgt * evg[lane]
                drows[e, pl.ds(0, 16)] = b0 * wgt
                drows[e, pl.ds(16, 16)] = b1 * wgt
                drows[e, pl.ds(32, 16)] = b2 * wgt
                drows[e, pl.ds(48, 16)] = b3 * wgt
            return 0

        lax.fori_loop(0, K // 16, group_body, 0)
        lanes = lax.iota(jnp.int32, 16)

        def clamp_body(g, _):
            v = sidx_v[pl.ds(g * 16, 16)]
            local = v - c * HALF
            pos = j * K + g * 16 + lanes
            ok = (local >= 0) & (local < HALF) & (pos < cnt)
            scat_v[pl.ds(g * 16, 16)] = jnp.where(ok, local, DUMMY_ROW)
            return 0

        lax.fori_loop(0, K // 16, clamp_body, 0)
        pltpu.sync_copy(drows, acc.at[scat_v], add=True)

    def process_list(w):
        pltpu.sync_copy(counts.at[c, w], cnt_v)
        cnt = cnt_v[pl.ds(0, 16)][0]
        nch = lax.div(cnt + (K - 1), K)
        npair = lax.div(nch + 1, 2)
        issue(w, 0, bufs[0])

        def pair_body(i, _):
            j = i * 2
            drain(bufs[0])
            issue(w, jnp.minimum(j + 1, nch), bufs[1])
            compute_scatter(bufs[0], j, cnt)
            drain(bufs[1])
            issue(w, jnp.minimum(j + 2, nch), bufs[0])
            compute_scatter(bufs[1], j + 1, cnt)
            return 0

        lax.fori_loop(0, npair, pair_body, 0)
        drain(bufs[0])

    process_list(2 * s)
    process_list(2 * s + 1)
    plsc.subcore_barrier()
    pltpu.sync_copy(acc.at[pl.ds(s * ROWS_PER_SUB, ROWS_PER_SUB)],
                    out.at[c, pl.ds(s * ROWS_PER_SUB, ROWS_PER_SUB)])


@jax.jit
def _route(srcp, dstp, evp):
    mesh = plsc.VectorSubcoreMesh(core_axis_name="c", subcore_axis_name="s")
    fn = pl.kernel(
        _route_body,
        mesh=mesh,
        compiler_params=pltpu.CompilerParams(use_tc_tiling_on_sc=False),
        out_type=(
            jax.ShapeDtypeStruct((2 * N_W * SUB_CAP,), jnp.int32),
            jax.ShapeDtypeStruct((2 * N_W * SUB_CAP,), jnp.int32),
            jax.ShapeDtypeStruct((2 * N_W * SUB_CAP,), jnp.float32),
            jax.ShapeDtypeStruct((2, N_W, 16), jnp.int32),
        ),
        scratch_types=(
            [pltpu.VMEM((K,), jnp.int32),
             pltpu.VMEM((K,), jnp.int32),
             pltpu.VMEM((K,), jnp.float32),
             pltpu.VMEM((K,), jnp.int32),
             pltpu.VMEM((K,), jnp.int32)] * RS
            + [pltpu.VMEM((16,), jnp.int32)]
            + [pltpu.SemaphoreType.DMA] * (2 * RS)
        ),
    )
    return fn(srcp, dstp, evp)


@jax.jit
def _edge_phase(emb, rsrc, rdst, rev, counts, zrows):
    mesh = plsc.VectorSubcoreMesh(core_axis_name="c", subcore_axis_name="s")
    fn = pl.kernel(
        _edge_body,
        mesh=mesh,
        compiler_params=pltpu.CompilerParams(use_tc_tiling_on_sc=False),
        out_type=jax.ShapeDtypeStruct((2, ACC_ROWS, D), jnp.float32),
        scratch_types=[
            pltpu.VMEM((K,), jnp.int32),
            pltpu.VMEM((K,), jnp.int32),
            pltpu.VMEM((K,), jnp.float32),
            pltpu.VMEM((K, D), jnp.float32),
            pltpu.VMEM((K, D), jnp.float32),
            pltpu.VMEM((K,), jnp.int32),
            pltpu.VMEM((K,), jnp.int32),
            pltpu.VMEM((K,), jnp.float32),
            pltpu.VMEM((K, D), jnp.float32),
            pltpu.VMEM((K, D), jnp.float32),
            pltpu.VMEM((K,), jnp.int32),
            pltpu.VMEM((16,), jnp.int32),
            pltpu.VMEM_SHARED((ACC_ROWS, D), jnp.float32),
            pltpu.SemaphoreType.DMA,
            pltpu.SemaphoreType.DMA,
            pltpu.SemaphoreType.DMA,
            pltpu.SemaphoreType.DMA,
        ],
    )
    return fn(emb, rsrc, rdst, rev, counts, zrows)


def kernel(user_emb, item_emb, edge_index, edge_values):
    all_emb = jnp.concatenate([user_emb, item_emb], axis=0)
    initial_emb = _l2norm(all_emb)

    pad = E_PAD - E
    srcp = jnp.concatenate([edge_index[0], jnp.zeros((pad,), jnp.int32)])
    dstp = jnp.concatenate([edge_index[1], jnp.zeros((pad,), jnp.int32)])
    evp = jnp.concatenate([edge_values, jnp.zeros((pad,), jnp.float32)])
    zrows = jnp.zeros((ROWS_PER_SUB, D), jnp.float32)

    rsrc, rdst, rev, counts = _route(srcp, dstp, evp)

    emb = all_emb
    emb_sum = all_emb
    for _ in range(3):
        emb = _l2norm(emb + RESIDUAL_COFF * initial_emb)
        acc = _edge_phase(emb, rsrc, rdst, rev, counts, zrows)
        neighbor = jnp.concatenate([acc[0, :HALF], acc[1, :HALF]], axis=0)
        emb = neighbor + RESIDUAL_COFF * (emb - initial_emb)
        emb_sum = emb_sum + emb
    light_out = emb_sum * 0.25
    return (light_out[:N_USERS], light_out[N_USERS:])


# speedup vs baseline: 3.9631x; 3.9631x over previous
import jax
import jax.numpy as jnp
from jax import lax
from jax.experimental import pallas as pl
from jax.experimental.pallas import tpu as pltpu
from jax.experimental.pallas import tpu_sc as plsc

N_USERS = 25000
N_ITEMS = 25000
N_NODES = N_USERS + N_ITEMS
D = 64
E = 800000
HALF = N_NODES // 2
N_SUB = 16
ROWS_PER_SUB = 1563
ACC_ROWS = N_SUB * ROWS_PER_SUB
DUMMY_ROW = 25004
K = 96
CHUNKS_PER_SUB = 522
EDGES_PER_SUB = CHUNKS_PER_SUB * K
E_PAD = N_SUB * EDGES_PER_SUB

RESIDUAL_COFF = 0.1

P0 = 7.8579951959925864
P1 = -22.224249412937251
P2 = 29.508311855171733
P3 = -24.651553647411124
P4 = 14.776720780182517
P5 = -6.8931074954875129


def _hsum_bcast(p):
    lanes = lax.iota(jnp.int32, 16)
    for sh in (8, 4, 2, 1):
        idx = (lanes + sh) & 15
        p = p + p.at[idx].get(mode="promise_in_bounds")
    return p


def _l2norm(x, eps=1e-12):
    n = jnp.linalg.norm(x, axis=-1, keepdims=True)
    return x / jnp.maximum(n, eps)


def _edge_body(emb, srci, dsti, ev, zrows, out,
               sidx0, didx0, ev0, srows0, drows0,
               sidx1, didx1, ev1, srows1, drows1,
               scat_v, acc, sa0, sb0, sa1, sb1):
    c = lax.axis_index("c")
    s = lax.axis_index("s")
    bufs = ((sidx0, didx0, ev0, srows0, drows0, sa0, sb0),
            (sidx1, didx1, ev1, srows1, drows1, sa1, sb1))

    pltpu.sync_copy(zrows, acc.at[pl.ds(s * ROWS_PER_SUB, ROWS_PER_SUB)])
    plsc.subcore_barrier()

    def issue(j, buf):
        sidx_v, didx_v, ev_v, srows, drows, sem_a, sem_b = buf
        base = (s * CHUNKS_PER_SUB + j) * K
        pltpu.sync_copy(srci.at[pl.ds(base, K)], sidx_v)
        pltpu.sync_copy(dsti.at[pl.ds(base, K)], didx_v)
        pltpu.sync_copy(ev.at[pl.ds(base, K)], ev_v)
        pltpu.async_copy(emb.at[sidx_v], srows, sem_a)
        pltpu.async_copy(emb.at[didx_v], drows, sem_b)

    def drain(buf):
        sidx_v, didx_v, ev_v, srows, drows, sem_a, sem_b = buf
        pltpu.make_async_copy(emb.at[sidx_v], srows, sem_a).wait()
        pltpu.make_async_copy(emb.at[didx_v], drows, sem_b).wait()

    def compute_scatter(buf):
        sidx_v, didx_v, ev_v, srows, drows, sem_a, sem_b = buf

        def group_body(g, _):
            evg = ev_v[pl.ds(g * 16, 16)]
            for lane in range(16):
                e = g * 16 + lane
                a0 = srows[e, pl.ds(0, 16)]
                a1 = srows[e, pl.ds(16, 16)]
                a2 = srows[e, pl.ds(32, 16)]
                a3 = srows[e, pl.ds(48, 16)]
                b0 = drows[e, pl.ds(0, 16)]
                b1 = drows[e, pl.ds(16, 16)]
                b2 = drows[e, pl.ds(32, 16)]
                b3 = drows[e, pl.ds(48, 16)]
                p = a0 * b0 + a1 * b1 + a2 * b2 + a3 * b3
                dot = _hsum_bcast(p)[0]
                cm = dot * (1.0 / 64.0)
                w = ((((P5 * cm + P4) * cm + P3) * cm + P2) * cm + P1) * cm + P0
                w = w * evg[lane]
                drows[e, pl.ds(0, 16)] = b0 * w
                drows[e, pl.ds(16, 16)] = b1 * w
                drows[e, pl.ds(32, 16)] = b2 * w
                drows[e, pl.ds(48, 16)] = b3 * w
            return 0

        lax.fori_loop(0, K // 16, group_body, 0)

        def clamp_body(g, _):
            v = sidx_v[pl.ds(g * 16, 16)]
            local = v - c * HALF
            ok = (local >= 0) & (local < HALF)
            scat_v[pl.ds(g * 16, 16)] = jnp.where(ok, local, DUMMY_ROW)
            return 0

        lax.fori_loop(0, K // 16, clamp_body, 0)
        pltpu.sync_copy(drows, acc.at[scat_v], add=True)

    last = CHUNKS_PER_SUB - 1
    issue(0, bufs[0])

    def pair_body(i, _):
        j = i * 2
        drain(bufs[0])
        issue(j + 1, bufs[1])
        compute_scatter(bufs[0])
        drain(bufs[1])
        issue(jnp.minimum(j + 2, last), bufs[0])
        compute_scatter(bufs[1])
        return 0

    lax.fori_loop(0, CHUNKS_PER_SUB // 2, pair_body, 0)
    drain(bufs[0])
    plsc.subcore_barrier()
    pltpu.sync_copy(acc.at[pl.ds(s * ROWS_PER_SUB, ROWS_PER_SUB)],
                    out.at[c, pl.ds(s * ROWS_PER_SUB, ROWS_PER_SUB)])


@jax.jit
def _edge_phase(emb, srcp, dstp, evp, zrows):
    mesh = plsc.VectorSubcoreMesh(core_axis_name="c", subcore_axis_name="s")
    fn = pl.kernel(
        _edge_body,
        mesh=mesh,
        compiler_params=pltpu.CompilerParams(use_tc_tiling_on_sc=False),
        out_type=jax.ShapeDtypeStruct((2, ACC_ROWS, D), jnp.float32),
        scratch_types=[
            pltpu.VMEM((K,), jnp.int32),
            pltpu.VMEM((K,), jnp.int32),
            pltpu.VMEM((K,), jnp.float32),
            pltpu.VMEM((K, D), jnp.float32),
            pltpu.VMEM((K, D), jnp.float32),
            pltpu.VMEM((K,), jnp.int32),
            pltpu.VMEM((K,), jnp.int32),
            pltpu.VMEM((K,), jnp.float32),
            pltpu.VMEM((K, D), jnp.float32),
            pltpu.VMEM((K, D), jnp.float32),
            pltpu.VMEM((K,), jnp.int32),
            pltpu.VMEM_SHARED((ACC_ROWS, D), jnp.float32),
            pltpu.SemaphoreType.DMA,
            pltpu.SemaphoreType.DMA,
            pltpu.SemaphoreType.DMA,
            pltpu.SemaphoreType.DMA,
        ],
    )
    return fn(emb, srcp, dstp, evp, zrows)


def kernel(user_emb, item_emb, edge_index, edge_values):
    all_emb = jnp.concatenate([user_emb, item_emb], axis=0)
    initial_emb = _l2norm(all_emb)

    pad = E_PAD - E
    srcp = jnp.concatenate([edge_index[0], jnp.zeros((pad,), jnp.int32)])
    dstp = jnp.concatenate([edge_index[1], jnp.zeros((pad,), jnp.int32)])
    evp = jnp.concatenate([edge_values, jnp.zeros((pad,), jnp.float32)])
    zrows = jnp.zeros((ROWS_PER_SUB, D), jnp.float32)

    emb = all_emb
    emb_sum = all_emb
    for _ in range(3):
        emb = _l2norm(emb + RESIDUAL_COFF * initial_emb)
        acc = _edge_phase(emb, srcp, dstp, evp, zrows)
        neighbor = jnp.concatenate([acc[0, :HALF], acc[1, :HALF]], axis=0)
        emb = neighbor + RESIDUAL_COFF * (emb - initial_emb)
        emb_sum = emb_sum + emb
    light_out = emb_sum * 0.25
    return (light_out[:N_USERS], light_out[N_USERS:])
